# baseline (speedup 1.0000x reference)
import functools

import jax
import jax.numpy as jnp
from jax import lax
from jax.experimental import pallas as pl
from jax.experimental.pallas import tpu as pltpu
from jax.experimental.pallas import tpu_sc as plsc

_B, _S, _D = 4, 8192, 1024
_NC, _NS = 2, 16
_NW = _NC * _NS
_SPW = _S // _NW
_CH = 16
_NCHUNK = _SPW // _CH
_UNROLL = 8

_mesh = plsc.VectorSubcoreMesh(core_axis_name="c", subcore_axis_name="s")


@functools.partial(
    pl.kernel,
    mesh=_mesh,
    out_type=jax.ShapeDtypeStruct((_B * _S, _D), jnp.float32),
    scratch_types=(
        [pltpu.VMEM((_CH, _D), jnp.float32) for _ in range(4)]
        + [pltpu.VMEM((_CH, _D), jnp.float32) for _ in range(2)]
        + [pltpu.SemaphoreType.DMA for _ in range(10)]
    ),
)
def _sc_add(x_hbm, pe_hbm, out_hbm,
            xb0, xb1, xb2, xb3, peb0, peb1,
            si0, si1, si2, si3, so0, so1, so2, so3, sp0, sp1):
    x_bufs = [xb0, xb1, xb2, xb3]
    pe_bufs = [peb0, peb1]
    in_sems = [si0, si1, si2, si3]
    out_sems = [so0, so1, so2, so3]
    pe_sems = [sp0, sp1]

    wid = lax.axis_index("s") * _NC + lax.axis_index("c")
    s_base = wid * _SPW

    steps = [(c, b) for c in range(_NCHUNK) for b in range(_B)]

    def x_row(c, b):
        return b * _S + s_base + c * _CH

    def load_x(c, b):
        return pltpu.async_copy(
            x_hbm.at[pl.ds(x_row(c, b), _CH)], x_bufs[b], in_sems[b])

    def load_pe(c):
        return pltpu.async_copy(
            pe_hbm.at[pl.ds(s_base + c * _CH, _CH)],
            pe_bufs[c % 2], pe_sems[c % 2])

    def store_x(c, b):
        return pltpu.async_copy(
            x_bufs[b], out_hbm.at[pl.ds(x_row(c, b), _CH)], out_sems[b])

    h_pe = [load_pe(0), None]
    h_in = [load_x(0, 0), load_x(0, 1), load_x(0, 2), None]
    h_out = [None, None, None, None]

    for g, (c, b) in enumerate(steps):
        h_in[b].wait()
        if b == 0:
            h_pe[c % 2].wait()
        if b == 1 and c + 1 < _NCHUNK:
            h_pe[(c + 1) % 2] = load_pe(c + 1)
        pe_v = pe_bufs[c % 2]
        x_v = x_bufs[b]

        def row_body(r, carry, x_v=x_v, pe_v=pe_v):
            def col_body(j, carry):
                base = j * 16 * _UNROLL
                for u in range(_UNROLL):
                    sl = pl.ds(base + u * 16, 16)
                    x_v[r, sl] = x_v[r, sl] + pe_v[r, sl]
                return carry

            return lax.fori_loop(0, _D // (16 * _UNROLL), col_body, carry)

        lax.fori_loop(0, _CH, row_body, 0)

        h_out[b] = store_x(c, b)
        if g + 3 < len(steps):
            nc, nb = steps[g + 3]
            if h_out[nb] is not None:
                h_out[nb].wait()
            h_in[nb] = load_x(nc, nb)

    for b in range(4):
        h_out[b].wait()


def kernel(x, pe_table):
    B, S, D = x.shape
    out = _sc_add(x.reshape(B * S, D), pe_table[:S])
    return out.reshape(B, S, D)


# speedup vs baseline: 3.1040x; 3.1040x over previous
import functools

import jax
import jax.numpy as jnp
from jax import lax
from jax.experimental import pallas as pl
from jax.experimental.pallas import tpu as pltpu
from jax.experimental.pallas import tpu_sc as plsc

_B, _S, _D = 4, 8192, 1024
_NC, _NS = 2, 16
_NW = _NC * _NS
_SPW = _S // _NW
_CH = 4
_NCHUNK = _SPW // _CH
_NSET = 4
_CU = 4

_mesh = plsc.VectorSubcoreMesh(core_axis_name="c", subcore_axis_name="s")


@functools.partial(
    pl.kernel,
    mesh=_mesh,
    out_type=jax.ShapeDtypeStruct((_B * _S, _D), jnp.float32),
    scratch_types=(
        [pltpu.VMEM((_CH, _D), jnp.float32) for _ in range(_NSET * _B)]
        + [pltpu.VMEM((_CH, _D), jnp.float32) for _ in range(_NSET)]
        + [pltpu.SemaphoreType.DMA for _ in range(_NSET * _B + _NSET)]
    ),
)
def _sc_add(x_hbm, pe_hbm, out_hbm, *scratch):
    x_bufs = [list(scratch[p * _B:(p + 1) * _B]) for p in range(_NSET)]
    pe_bufs = list(scratch[_NSET * _B:_NSET * _B + _NSET])
    sems = scratch[_NSET * _B + _NSET:]
    x_sems = [list(sems[p * _B:(p + 1) * _B]) for p in range(_NSET)]
    pe_sems = list(sems[_NSET * _B:_NSET * _B + _NSET])

    wid = lax.axis_index("s") * _NC + lax.axis_index("c")
    s_base = wid * _SPW

    def pe_row(c):
        return s_base + c * _CH

    def x_row(c, b):
        return b * _S + pe_row(c)

    def issue_loads(c, p):
        pltpu.async_copy(
            pe_hbm.at[pl.ds(pe_row(c), _CH)], pe_bufs[p], pe_sems[p])
        for b in range(_B):
            pltpu.async_copy(
                x_hbm.at[pl.ds(x_row(c, b), _CH)], x_bufs[p][b], x_sems[p][b])

    def wait_loads(p):
        pltpu.make_async_copy(
            pe_hbm.at[pl.ds(0, _CH)], pe_bufs[p], pe_sems[p]).wait()
        for b in range(_B):
            pltpu.make_async_copy(
                x_hbm.at[pl.ds(0, _CH)], x_bufs[p][b], x_sems[p][b]).wait()

    def issue_stores(c, p):
        for b in range(_B):
            pltpu.async_copy(
                x_bufs[p][b], out_hbm.at[pl.ds(x_row(c, b), _CH)],
                x_sems[p][b])

    def wait_stores(p):
        for b in range(_B):
            pltpu.make_async_copy(
                x_bufs[p][b], out_hbm.at[pl.ds(0, _CH)], x_sems[p][b]).wait()

    def compute(p):
        for r in range(_CH):
            def col_body(j, carry, r=r, p=p):
                base = j * 16 * _CU
                for u in range(_CU):
                    sl = pl.ds(base + u * 16, 16)
                    v = pe_bufs[p][r, sl]
                    for b in range(_B):
                        x_bufs[p][b][r, sl] = x_bufs[p][b][r, sl] + v
                return carry

            lax.fori_loop(0, _D // (16 * _CU), col_body, 0)

    issue_loads(0, 0)
    issue_loads(1, 1)

    for j, c in enumerate(range(_NSET)):
        p, p2 = j, (j + 2) % _NSET
        if j < 2:
            issue_loads(c + 2, p2)
        else:
            wait_stores(p2)
            issue_loads(c + 2, p2)
        wait_loads(p)
        compute(p)
        issue_stores(c, p)

    def rotation(cp, carry):
        for j in range(_NSET):
            c = cp * _NSET + j
            p, p2 = j, (j + 2) % _NSET
            wait_stores(p2)
            issue_loads(c + 2, p2)
            wait_loads(p)
            compute(p)
            issue_stores(c, p)
        return carry

    lax.fori_loop(1, _NCHUNK // _NSET - 1, rotation, 0)

    for j in range(_NSET):
        c = (_NCHUNK - _NSET) + j
        p, p2 = j, (j + 2) % _NSET
        if c + 2 < _NCHUNK:
            wait_stores(p2)
            issue_loads(c + 2, p2)
        wait_loads(p)
        compute(p)
        issue_stores(c, p)

    for p in range(_NSET):
        wait_stores(p)


def kernel(x, pe_table):
    B, S, D = x.shape
    out = _sc_add(x.reshape(B * S, D), pe_table[:S])
    return out.reshape(B, S, D)
